# baseline (speedup 1.0000x reference)
import jax
import jax.numpy as jnp
from jax.experimental import pallas as pl
import jax.experimental.pallas.tpu as pltpu

N = 10000
D_IN = 256
D_OUT = 256
BM = 400


def _fused_kernel(f_ref, w_ref, adj_a_ref, adj_b_ref, o_ref, sup_ref):
    @pl.when(pl.program_id(0) == 0)
    def _():
        sup = jnp.dot(f_ref[...], w_ref[...],
                      preferred_element_type=jnp.float32)
        sup_ref[...] = sup.astype(jnp.bfloat16)

    o_ref[: BM // 2, :] = jnp.dot(
        adj_a_ref[...].astype(jnp.bfloat16), sup_ref[...],
        preferred_element_type=jnp.float32)
    o_ref[BM // 2 :, :] = jnp.dot(
        adj_b_ref[...].astype(jnp.bfloat16), sup_ref[...],
        preferred_element_type=jnp.float32)


def kernel(features, adj, weight):
    return pl.pallas_call(
        _fused_kernel,
        grid=(pl.cdiv(N, BM),),
        in_specs=[
            pl.BlockSpec((N, D_IN), lambda i: (0, 0)),
            pl.BlockSpec((D_IN, D_OUT), lambda i: (0, 0)),
            pl.BlockSpec((BM // 2, N), lambda i: (2 * i, 0)),
            pl.BlockSpec((BM // 2, N), lambda i: (2 * i + 1, 0)),
        ],
        out_specs=pl.BlockSpec((BM, D_OUT), lambda i: (i, 0)),
        out_shape=jax.ShapeDtypeStruct((N, D_OUT), jnp.float32),
        scratch_shapes=[pltpu.VMEM((N, D_OUT), jnp.bfloat16)],
    )(features, weight, adj, adj)


# speedup vs baseline: 1.0162x; 1.0162x over previous
import jax
import jax.numpy as jnp
from jax.experimental import pallas as pl
import jax.experimental.pallas.tpu as pltpu

N = 10000
D_IN = 256
D_OUT = 256
BM = 400


def _fused_kernel(f_ref, w_ref, adj_ref, o_ref, sup_ref):
    @pl.when(pl.program_id(0) == 0)
    def _():
        sup_ref[...] = jnp.dot(f_ref[...], w_ref[...],
                               preferred_element_type=jnp.float32)

    o_ref[...] = jnp.dot(adj_ref[...], sup_ref[...],
                         preferred_element_type=jnp.float32)


def kernel(features, adj, weight):
    return pl.pallas_call(
        _fused_kernel,
        grid=(pl.cdiv(N, BM),),
        in_specs=[
            pl.BlockSpec((N, D_IN), lambda i: (0, 0)),
            pl.BlockSpec((D_IN, D_OUT), lambda i: (0, 0)),
            pl.BlockSpec((BM, N), lambda i: (i, 0)),
        ],
        out_specs=pl.BlockSpec((BM, D_OUT), lambda i: (i, 0)),
        out_shape=jax.ShapeDtypeStruct((N, D_OUT), jnp.float32),
        scratch_shapes=[pltpu.VMEM((N, D_OUT), jnp.float32)],
    )(features, weight, adj)


# speedup vs baseline: 1.0189x; 1.0026x over previous
import jax
import jax.numpy as jnp
from jax.experimental import pallas as pl

N = 10000
D_IN = 256
D_OUT = 256
BM = 400


def _fused_kernel(f_ref, w_ref, adj_ref, o_ref):
    agg = jnp.dot(adj_ref[...], f_ref[...],
                  preferred_element_type=jnp.float32)
    o_ref[...] = jnp.dot(agg, w_ref[...],
                         preferred_element_type=jnp.float32)


def kernel(features, adj, weight):
    return pl.pallas_call(
        _fused_kernel,
        grid=(pl.cdiv(N, BM),),
        in_specs=[
            pl.BlockSpec((N, D_IN), lambda i: (0, 0)),
            pl.BlockSpec((D_IN, D_OUT), lambda i: (0, 0)),
            pl.BlockSpec((BM, N), lambda i: (i, 0)),
        ],
        out_specs=pl.BlockSpec((BM, D_OUT), lambda i: (i, 0)),
        out_shape=jax.ShapeDtypeStruct((N, D_OUT), jnp.float32),
    )(features, weight, adj)
